# Initial kernel scaffold; baseline (speedup 1.0000x reference)
import jax
import jax.numpy as jnp
from jax.experimental import pallas as pl


def kernel(feat, edge_index, W, b):
    raise NotImplementedError("write your pallas kernel here")


# speedup vs baseline: 2.3007x; 2.3007x over previous
import functools

import jax
import jax.numpy as jnp
from jax import lax
from jax.experimental import pallas as pl
from jax.experimental.pallas import tpu as pltpu
from jax.experimental.pallas import tpu_sc as plsc

NC, NS, L = 2, 16, 16
N = 10000
E = 160000
D = 256
H = D // NC
RS = 640
NP = NS * RS
SB = 160
CB = 128
EPT = 10240
ET = NS * EPT
EPW = ET // (NC * NS)

_f32 = jnp.float32
_i32 = jnp.int32


def _deg_body(didx, deg2, degsh, di, ones, zb, stg):
    c = lax.axis_index("c")
    s = lax.axis_index("s")

    for j in range(CB // L):
        ones[pl.ds(j * L, L)] = jnp.ones((L,), _f32)

    def _z(t, _):
        zb[pl.ds(t * L, L)] = jnp.zeros((L,), _f32)
        return 0

    lax.fori_loop(0, RS // L, _z, 0)
    pltpu.sync_copy(zb, degsh.at[pl.ds(s * RS, RS)])
    plsc.subcore_barrier()

    w = c * NS + s

    def _e(i, _):
        base = w * EPW + i * CB
        pltpu.sync_copy(didx.at[pl.ds(base, CB)], di)
        pltpu.sync_copy(ones, degsh.at[di], add=True)
        return 0

    lax.fori_loop(0, EPW // CB, _e, 0)
    plsc.subcore_barrier()

    pltpu.sync_copy(degsh.at[pl.ds(s * RS, RS)], stg)
    pltpu.sync_copy(stg, deg2.at[pl.ds(c * NP + s * RS, RS)])


_deg_call = functools.partial(
    pl.kernel,
    out_type=jax.ShapeDtypeStruct((NC * NP,), _f32),
    mesh=plsc.VectorSubcoreMesh(core_axis_name="c", subcore_axis_name="s"),
    scratch_types=[
        pltpu.VMEM_SHARED((NP,), _f32),
        pltpu.VMEM((CB,), _i32),
        pltpu.VMEM((CB,), _f32),
        pltpu.VMEM((RS,), _f32),
        pltpu.VMEM((RS,), _f32),
    ],
)(_deg_body)


def _hop_body(sin, gidx, sidx, norm2x, sout,
              agg, gi_raw, gi_v, si_v, rows, n2st):
    c = lax.axis_index("c")
    s = lax.axis_index("s")
    coff = c * NP
    row0 = s * RS

    def _zrow(r, _):
        for j in range(H // L):
            rows[r, pl.ds(j * L, L)] = jnp.zeros((L,), _f32)
        return 0

    lax.fori_loop(0, CB, _zrow, 0)
    for t in range(RS // CB):
        pltpu.sync_copy(rows, agg.at[pl.ds(row0 + t * CB, CB)])
    plsc.subcore_barrier()

    def _edge(i, _):
        base = s * EPT + i * CB
        pltpu.sync_copy(gidx.at[pl.ds(base, CB)], gi_raw)
        pltpu.sync_copy(sidx.at[pl.ds(base, CB)], si_v)
        for j in range(CB // L):
            gi_v[pl.ds(j * L, L)] = gi_raw[pl.ds(j * L, L)] + coff
        pltpu.sync_copy(sin.at[gi_v], rows)
        pltpu.sync_copy(rows, agg.at[si_v], add=True)
        return 0

    lax.fori_loop(0, EPT // CB, _edge, 0)
    plsc.subcore_barrier()

    for t in range(RS // CB):
        pltpu.sync_copy(agg.at[pl.ds(row0 + t * CB, CB)], rows)
        pltpu.sync_copy(norm2x.at[pl.ds(row0 + t * CB, CB)], n2st)

        def _srow(r, _):
            for j in range(H // L):
                rows[r, pl.ds(j * L, L)] = (
                    rows[r, pl.ds(j * L, L)] * n2st[r, pl.ds(j * L, L)]
                )
            return 0

        lax.fori_loop(0, CB, _srow, 0)
        pltpu.sync_copy(rows, sout.at[pl.ds(coff + row0 + t * CB, CB)])


_hop_call = functools.partial(
    pl.kernel,
    out_type=jax.ShapeDtypeStruct((NC * NP, H), _f32),
    mesh=plsc.VectorSubcoreMesh(core_axis_name="c", subcore_axis_name="s"),
    scratch_types=[
        pltpu.VMEM_SHARED((NP, H), _f32),
        pltpu.VMEM((CB,), _i32),
        pltpu.VMEM((CB,), _i32),
        pltpu.VMEM((CB,), _i32),
        pltpu.VMEM((CB, H), _f32),
        pltpu.VMEM((CB, H), _f32),
    ],
)(_hop_body)


def _prep_body(feat_ref, deg2_ref, s0_ref, n2x_ref, rn_ref):
    d = deg2_ref[0:NP, :] + deg2_ref[NP : 2 * NP, :]
    cl = jnp.maximum(d, 1.0)
    norm = lax.rsqrt(cl)
    n2x_ref[...] = jnp.broadcast_to(1.0 / cl, (NP, H))
    rn_ref[...] = jnp.sqrt(cl)
    s0_ref[0:NP, :] = feat_ref[:, 0:H] * norm
    s0_ref[NP : 2 * NP, :] = feat_ref[:, H : 2 * H] * norm


_prep_call = pl.pallas_call(
    _prep_body,
    out_shape=[
        jax.ShapeDtypeStruct((NC * NP, H), _f32),
        jax.ShapeDtypeStruct((NP, H), _f32),
        jax.ShapeDtypeStruct((NP, 1), _f32),
    ],
)


def _mm_body(feat_ref, w0_ref, wh_ref, rn_ref, b_ref, *rest):
    s_refs = rest[:8]
    out_ref = rest[8]
    acc = jnp.zeros((RS, D), _f32)
    for k in range(8):
        for c in range(NC):
            acc = acc + jnp.dot(s_refs[k][c], wh_ref[k, c],
                                preferred_element_type=_f32)
    base = jnp.dot(feat_ref[...], w0_ref[...], preferred_element_type=_f32)
    out_ref[...] = base + rn_ref[...] * acc + b_ref[...]


_mm_call = pl.pallas_call(
    _mm_body,
    grid=(NS,),
    in_specs=[
        pl.BlockSpec((RS, D), lambda i: (i, 0)),
        pl.BlockSpec((D, D), lambda i: (0, 0)),
        pl.BlockSpec((8, NC, H, D), lambda i: (0, 0, 0, 0)),
        pl.BlockSpec((RS, 1), lambda i: (i, 0)),
        pl.BlockSpec((1, D), lambda i: (0, 0)),
    ] + [pl.BlockSpec((NC, RS, H), lambda i: (0, i, 0)) for _ in range(8)],
    out_specs=pl.BlockSpec((RS, D), lambda i: (i, 0)),
    out_shape=jax.ShapeDtypeStruct((NP, D), _f32),
)


def kernel(feat, edge_index, W, b):
    src = edge_index[0].astype(_i32)
    dst = edge_index[1].astype(_i32)

    pad = ET - E
    zpad = jnp.zeros((pad,), _i32)
    npad = jnp.full((pad,), N, _i32)
    dst_g = jnp.concatenate([dst, zpad])
    dst_s = jnp.concatenate([dst, npad])
    src_g = jnp.concatenate([src, zpad])
    src_s = jnp.concatenate([src, npad])

    featp = jnp.pad(feat, ((0, NP - N), (0, 0)))

    deg2 = _deg_call(dst_s)
    s0, n2x, rn = _prep_call(featp, deg2.reshape(NC * NP, 1))

    s_list = []
    s_cur = s0
    for k in range(8):
        g, sc = (dst_g, src_s) if k < 4 else (src_g, dst_s)
        s_cur = _hop_call(s_cur, g, sc, n2x)
        s_list.append(s_cur)

    w0 = W[0:D]
    wh = W[D:].reshape(8, NC, H, D)
    b2 = b.reshape(1, D)
    s3d = [sk.reshape(NC, NP, H) for sk in s_list]
    outp = _mm_call(featp, w0, wh, rn, b2, *s3d)
    return outp[:N]


# speedup vs baseline: 3.1007x; 1.3477x over previous
import functools

import jax
import jax.numpy as jnp
from jax import lax
from jax.experimental import pallas as pl
from jax.experimental.pallas import tpu as pltpu
from jax.experimental.pallas import tpu_sc as plsc

NC, NS, L = 2, 16, 16
N = 10000
E = 160000
D = 256
H = D // NC
RS = 640
NP = NS * RS
CB = 128
DC = 64
EPT = 10240
NCH = EPT // CB
ET = NS * EPT
DCB = 128
EPW = ET // (NC * NS)
DNCH = EPW // DCB

_f32 = jnp.float32
_i32 = jnp.int32


def _deg_body(darr, deg2, degsh, di_all, ones, zb, stg, sem):
    c = lax.axis_index("c")
    s = lax.axis_index("s")
    w = c * NS + s

    for j in range(DCB // L):
        ones[pl.ds(j * L, L)] = jnp.ones((L,), _f32)

    def _z(t, _):
        zb[pl.ds(t * L, L)] = jnp.zeros((L,), _f32)
        return 0

    lax.fori_loop(0, RS // L, _z, 0)
    pltpu.sync_copy(zb, degsh.at[pl.ds(s * RS, RS)])
    pltpu.sync_copy(darr.at[w], di_all)
    plsc.subcore_barrier()

    def _grp(g, _):
        for u in range(8):
            pltpu.async_copy(ones, degsh.at[di_all.at[g * 8 + u]], sem, add=True)
        for u in range(8):
            pltpu.make_async_copy(ones, degsh.at[di_all.at[0]], sem).wait()
        return 0

    lax.fori_loop(0, DNCH // 8, _grp, 0)
    plsc.subcore_barrier()

    pltpu.sync_copy(degsh.at[pl.ds(s * RS, RS)], stg)
    pltpu.sync_copy(stg, deg2.at[pl.ds(c * NP + s * RS, RS)])


_deg_call = functools.partial(
    pl.kernel,
    out_type=jax.ShapeDtypeStruct((NC * NP,), _f32),
    mesh=plsc.VectorSubcoreMesh(core_axis_name="c", subcore_axis_name="s"),
    scratch_types=[
        pltpu.VMEM_SHARED((NP,), _f32),
        pltpu.VMEM((DNCH, DCB), _i32),
        pltpu.VMEM((DCB,), _f32),
        pltpu.VMEM((RS,), _f32),
        pltpu.VMEM((RS,), _f32),
        pltpu.SemaphoreType.DMA,
    ],
)(_deg_body)


def _hop_body(sin, garr, sarr, norm2x, sout,
              agg, xg0, xg1, xg2, xg3, xs0, xs1, xs2, xs3, r0, r1,
              gA, gB, sA, sB, pA, pB, pC, pD):
    c = lax.axis_index("c")
    s = lax.axis_index("s")
    w = c * NS + s
    coff = c * NP
    row0 = s * RS

    rows = [r0, r1]
    ixg = [xg0, xg1, xg2, xg3]
    ixs = [xs0, xs1, xs2, xs3]
    gsem = [gA, gB]
    ssem = [sA, sB]
    psem = [pA, pB, pC, pD]

    def ix_start(i, q):
        pltpu.async_copy(garr.at[w * NCH + i], ixg[q], psem[q])
        pltpu.async_copy(sarr.at[s * NCH + i], ixs[q], psem[q])

    def ix_wait(q):
        pltpu.make_async_copy(garr.at[0], ixg[q], psem[q]).wait()
        pltpu.make_async_copy(sarr.at[0], ixs[q], psem[q]).wait()

    def g_start(b, q):
        pltpu.async_copy(sin.at[ixg[q]], rows[b], gsem[b])

    def g_wait(b):
        pltpu.make_async_copy(sin.at[ixg[0]], rows[b], gsem[b]).wait()

    def s_start(b, q):
        pltpu.async_copy(rows[b], agg.at[ixs[q]], ssem[b], add=True)

    def s_wait(b):
        pltpu.make_async_copy(rows[b], agg.at[ixs[0]], ssem[b]).wait()

    def _zrow(r, _):
        for j in range(H // L):
            r0[r, pl.ds(j * L, L)] = jnp.zeros((L,), _f32)
        return 0

    lax.fori_loop(0, CB, _zrow, 0)
    for t in range(RS // CB):
        pltpu.async_copy(r0, agg.at[pl.ds(row0 + t * CB, CB)], sA)
    for t in range(RS // CB):
        pltpu.make_async_copy(r0, agg.at[pl.ds(row0, CB)], sA).wait()
    plsc.subcore_barrier()

    pltpu.sync_copy(garr.at[w * NCH], ixg[0])
    pltpu.sync_copy(sarr.at[s * NCH], ixs[0])
    g_start(0, 0)
    ix_start(1, 1)
    g_wait(0)
    s_start(0, 0)
    ix_wait(1)
    g_start(1, 1)

    def _grp(t, _):
        for u in range(4):
            i = t * 4 + u
            b = u % 2
            q = u
            qn = (u + 1) % 4
            ix_start(i + 1, qn)
            g_wait(b)
            s_start(b, q)
            s_wait(1 - b)
            ix_wait(qn)
            g_start(1 - b, qn)
        return 0

    for u in range(1, 4):
        b = u % 2
        qn = (u + 1) % 4
        ix_start(u + 1, qn)
        g_wait(b)
        s_start(b, u)
        s_wait(1 - b)
        ix_wait(qn)
        g_start(1 - b, qn)

    lax.fori_loop(1, NCH // 4 - 1, _grp, 0)

    base = NCH - 4
    for u in range(4):
        i = base + u
        b = u % 2
        if u < 3:
            ix_start(i + 1, (u + 1) % 4)
            g_wait(b)
            s_start(b, u)
            s_wait(1 - b)
            ix_wait((u + 1) % 4)
            g_start(1 - b, (u + 1) % 4)
        else:
            g_wait(b)
            s_start(b, u)
            s_wait(1 - b)
    s_wait(1)
    plsc.subcore_barrier()

    for t in range(RS // DC):
        p = t % 2
        db = rows[p]
        if t >= 2:
            pltpu.make_async_copy(
                db.at[pl.ds(0, DC)], sout.at[pl.ds(coff, DC)], ssem[p]).wait()
        pltpu.async_copy(agg.at[pl.ds(row0 + t * DC, DC)], db.at[pl.ds(0, DC)],
                         gsem[p])
        pltpu.async_copy(norm2x.at[pl.ds(row0 + t * DC, DC)],
                         db.at[pl.ds(DC, DC)], psem[p])
        pltpu.make_async_copy(agg.at[pl.ds(row0, DC)], db.at[pl.ds(0, DC)],
                              gsem[p]).wait()
        pltpu.make_async_copy(norm2x.at[pl.ds(row0, DC)], db.at[pl.ds(DC, DC)],
                              psem[p]).wait()

        def _srow(r, _):
            for j in range(H // L):
                db[r, pl.ds(j * L, L)] = (
                    db[r, pl.ds(j * L, L)] * db[r + DC, pl.ds(j * L, L)]
                )
            return 0

        lax.fori_loop(0, DC, _srow, 0)
        pltpu.async_copy(db.at[pl.ds(0, DC)],
                         sout.at[pl.ds(coff + row0 + t * DC, DC)], ssem[p])
    for p in range(2):
        pltpu.make_async_copy(
            rows[p].at[pl.ds(0, DC)], sout.at[pl.ds(coff, DC)], ssem[p]).wait()


_hop_call = functools.partial(
    pl.kernel,
    out_type=jax.ShapeDtypeStruct((NC * NP, H), _f32),
    mesh=plsc.VectorSubcoreMesh(core_axis_name="c", subcore_axis_name="s"),
    scratch_types=[
        pltpu.VMEM_SHARED((NP, H), _f32),
        pltpu.VMEM((CB,), _i32),
        pltpu.VMEM((CB,), _i32),
        pltpu.VMEM((CB,), _i32),
        pltpu.VMEM((CB,), _i32),
        pltpu.VMEM((CB,), _i32),
        pltpu.VMEM((CB,), _i32),
        pltpu.VMEM((CB,), _i32),
        pltpu.VMEM((CB,), _i32),
        pltpu.VMEM((CB, H), _f32),
        pltpu.VMEM((CB, H), _f32),
        pltpu.SemaphoreType.DMA,
        pltpu.SemaphoreType.DMA,
        pltpu.SemaphoreType.DMA,
        pltpu.SemaphoreType.DMA,
        pltpu.SemaphoreType.DMA,
        pltpu.SemaphoreType.DMA,
        pltpu.SemaphoreType.DMA,
        pltpu.SemaphoreType.DMA,
    ],
)(_hop_body)


def _prep_body(feat_ref, deg2_ref, s0_ref, n2x_ref, rn_ref):
    d = deg2_ref[0:NP, :] + deg2_ref[NP : 2 * NP, :]
    cl = jnp.maximum(d, 1.0)
    norm = lax.rsqrt(cl)
    n2x_ref[...] = jnp.broadcast_to(1.0 / cl, (NP, H))
    rn_ref[...] = jnp.sqrt(cl)
    s0_ref[0:NP, :] = feat_ref[:, 0:H] * norm
    s0_ref[NP : 2 * NP, :] = feat_ref[:, H : 2 * H] * norm


_prep_call = pl.pallas_call(
    _prep_body,
    out_shape=[
        jax.ShapeDtypeStruct((NC * NP, H), _f32),
        jax.ShapeDtypeStruct((NP, H), _f32),
        jax.ShapeDtypeStruct((NP, 1), _f32),
    ],
)


def _mm_body(feat_ref, w0_ref, wh_ref, rn_ref, b_ref, *rest):
    s_refs = rest[:8]
    out_ref = rest[8]
    acc = jnp.zeros((RS, D), _f32)
    for k in range(8):
        for c in range(NC):
            acc = acc + jnp.dot(s_refs[k][c], wh_ref[k, c],
                                preferred_element_type=_f32)
    base = jnp.dot(feat_ref[...], w0_ref[...], preferred_element_type=_f32)
    out_ref[...] = base + rn_ref[...] * acc + b_ref[...]


_mm_call = pl.pallas_call(
    _mm_body,
    grid=(NS,),
    in_specs=[
        pl.BlockSpec((RS, D), lambda i: (i, 0)),
        pl.BlockSpec((D, D), lambda i: (0, 0)),
        pl.BlockSpec((8, NC, H, D), lambda i: (0, 0, 0, 0)),
        pl.BlockSpec((RS, 1), lambda i: (i, 0)),
        pl.BlockSpec((1, D), lambda i: (0, 0)),
    ] + [pl.BlockSpec((NC, RS, H), lambda i: (0, i, 0)) for _ in range(8)],
    out_specs=pl.BlockSpec((RS, D), lambda i: (i, 0)),
    out_shape=jax.ShapeDtypeStruct((NP, D), _f32),
)


def kernel(feat, edge_index, W, b):
    src = edge_index[0].astype(_i32)
    dst = edge_index[1].astype(_i32)

    pad = ET - E
    zpad = jnp.zeros((pad,), _i32)
    npad = jnp.full((pad,), N, _i32)
    dst_g = jnp.concatenate([dst, zpad]).reshape(NS * NCH, CB)
    dst_s = jnp.concatenate([dst, npad]).reshape(NS * NCH, CB)
    src_g = jnp.concatenate([src, zpad]).reshape(NS * NCH, CB)
    src_s = jnp.concatenate([src, npad]).reshape(NS * NCH, CB)
    dst_g2 = jnp.concatenate([dst_g, dst_g + NP], axis=0)
    src_g2 = jnp.concatenate([src_g, src_g + NP], axis=0)
    darr = dst_s.reshape(NC * NS, DNCH, DCB)

    featp = jnp.pad(feat, ((0, NP - N), (0, 0)))

    deg2 = _deg_call(darr)
    s0, n2x, rn = _prep_call(featp, deg2.reshape(NC * NP, 1))

    s_list = []
    s_cur = s0
    for k in range(8):
        g, sc = (dst_g2, src_s) if k < 4 else (src_g2, dst_s)
        s_cur = _hop_call(s_cur, g, sc, n2x)
        s_list.append(s_cur)

    w0 = W[0:D]
    wh = W[D:].reshape(8, NC, H, D)
    b2 = b.reshape(1, D)
    s3d = [sk.reshape(NC, NP, H) for sk in s_list]
    outp = _mm_call(featp, w0, wh, rn, b2, *s3d)
    return outp[:N]


# speedup vs baseline: 3.2638x; 1.0526x over previous
import functools

import jax
import jax.numpy as jnp
from jax import lax
from jax.experimental import pallas as pl
from jax.experimental.pallas import tpu as pltpu
from jax.experimental.pallas import tpu_sc as plsc

NC, NS, L = 2, 16, 16
N = 10000
E = 160000
D = 256
H = D // NC
RS = 640
NP = NS * RS
CB = 64
EPT = 10240
NCH = EPT // CB
ET = NS * EPT
DCB = 128
EPW = ET // (NC * NS)
DNCH = EPW // DCB

_f32 = jnp.float32
_i32 = jnp.int32


def _deg_body(darr, deg2, degsh, di_all, ones, zb, stg, sem):
    c = lax.axis_index("c")
    s = lax.axis_index("s")
    w = c * NS + s

    for j in range(DCB // L):
        ones[pl.ds(j * L, L)] = jnp.ones((L,), _f32)

    def _z(t, _):
        zb[pl.ds(t * L, L)] = jnp.zeros((L,), _f32)
        return 0

    lax.fori_loop(0, RS // L, _z, 0)
    pltpu.sync_copy(zb, degsh.at[pl.ds(s * RS, RS)])
    pltpu.sync_copy(darr.at[w], di_all)
    plsc.subcore_barrier()

    def _grp(g, _):
        for u in range(8):
            pltpu.async_copy(ones, degsh.at[di_all.at[g * 8 + u]], sem, add=True)
        for u in range(8):
            pltpu.make_async_copy(ones, degsh.at[di_all.at[0]], sem).wait()
        return 0

    lax.fori_loop(0, DNCH // 8, _grp, 0)
    plsc.subcore_barrier()

    pltpu.sync_copy(degsh.at[pl.ds(s * RS, RS)], stg)
    pltpu.sync_copy(stg, deg2.at[pl.ds(c * NP + s * RS, RS)])


_deg_call = functools.partial(
    pl.kernel,
    out_type=jax.ShapeDtypeStruct((NC * NP,), _f32),
    mesh=plsc.VectorSubcoreMesh(core_axis_name="c", subcore_axis_name="s"),
    scratch_types=[
        pltpu.VMEM_SHARED((NP,), _f32),
        pltpu.VMEM((DNCH, DCB), _i32),
        pltpu.VMEM((DCB,), _f32),
        pltpu.VMEM((RS,), _f32),
        pltpu.VMEM((RS,), _f32),
        pltpu.SemaphoreType.DMA,
    ],
)(_deg_body)


def _hops_body(s0, garr, sarr, norm2x,
               o1, o2, o3, o4,
               agg, xg0, xg1, xg2, xg3, xg4, xg5, xg6, xg7,
               xs0, xs1, xs2, xs3, xs4, xs5, xs6, xs7,
               r0, r1, r2, r3,
               gA, gB, gC, gD, sA, sB, sC, sD,
               pA, pB, pC, pD, pE, pF, pG, pH):
    c = lax.axis_index("c")
    s = lax.axis_index("s")
    w = c * NS + s
    coff = c * NP
    row0 = s * RS

    rows = [r0, r1, r2, r3]
    ixg = [xg0, xg1, xg2, xg3, xg4, xg5, xg6, xg7]
    ixs = [xs0, xs1, xs2, xs3, xs4, xs5, xs6, xs7]
    gsem = [gA, gB, gC, gD]
    ssem = [sA, sB, sC, sD]
    psem = [pA, pB, pC, pD, pE, pF, pG, pH]

    def one_hop(sin, sout, garr, sarr):
        def ix_start(i, q):
            pltpu.async_copy(garr.at[w * NCH + i], ixg[q], psem[q])
            pltpu.async_copy(sarr.at[s * NCH + i], ixs[q], psem[q])

        def ix_wait(q):
            pltpu.make_async_copy(garr.at[0], ixg[q], psem[q]).wait()
            pltpu.make_async_copy(sarr.at[0], ixs[q], psem[q]).wait()

        def g_start(b, q):
            pltpu.async_copy(sin.at[ixg[q]], rows[b], gsem[b])

        def g_wait(b):
            pltpu.make_async_copy(sin.at[ixg[0]], rows[b], gsem[b]).wait()

        def s_start(b, q):
            pltpu.async_copy(rows[b], agg.at[ixs[q]], ssem[b], add=True)

        def s_wait(b):
            pltpu.make_async_copy(rows[b], agg.at[ixs[0]], ssem[b]).wait()

        def _zrow(r, _):
            for j in range(H // L):
                r0[r, pl.ds(j * L, L)] = jnp.zeros((L,), _f32)
            return 0

        lax.fori_loop(0, CB, _zrow, 0)
        for t in range(RS // CB):
            pltpu.async_copy(r0, agg.at[pl.ds(row0 + t * CB, CB)], sA)
        for t in range(RS // CB):
            pltpu.make_async_copy(r0, agg.at[pl.ds(row0, CB)], sA).wait()
        plsc.subcore_barrier()

        for q in range(3):
            pltpu.sync_copy(garr.at[w * NCH + q], ixg[q])
            pltpu.sync_copy(sarr.at[s * NCH + q], ixs[q])
        ix_start(3, 3)
        g_start(0, 0)
        g_start(1, 1)
        g_start(2, 2)
        ix_start(4, 4)
        g_wait(0)
        s_start(0, 0)
        ix_wait(3)
        g_start(3, 3)
        for i in range(1, 8):
            b = i % 4
            ix_start(i + 4, (i + 4) % 8)
            g_wait(b)
            s_start(b, i % 8)
            s_wait((i - 1) % 4)
            ix_wait((i + 3) % 8)
            g_start((i + 3) % 4, (i + 3) % 8)

        def _grp(t, _):
            for u in range(8):
                i = t * 8 + u
                b = u % 4
                ix_start(i + 4, (u + 4) % 8)
                g_wait(b)
                s_start(b, u)
                s_wait((u - 1) % 4)
                ix_wait((u + 3) % 8)
                g_start((u + 3) % 4, (u + 3) % 8)
            return 0

        lax.fori_loop(1, NCH // 8 - 1, _grp, 0)

        base = NCH - 8
        for u in range(8):
            i = base + u
            b = u % 4
            if i + 4 < NCH:
                ix_start(i + 4, (u + 4) % 8)
            g_wait(b)
            s_start(b, u)
            s_wait((u - 1) % 4)
            if i + 3 < NCH:
                ix_wait((u + 3) % 8)
                g_start((u + 3) % 4, (u + 3) % 8)
        s_wait(3)
        plsc.subcore_barrier()

        for t in range(RS // CB):
            p = t % 2
            db, nb = rows[2 * p], rows[2 * p + 1]
            if t >= 2:
                pltpu.make_async_copy(
                    db, sout.at[pl.ds(coff, CB)], ssem[2 * p]).wait()
            pltpu.async_copy(agg.at[pl.ds(row0 + t * CB, CB)], db, gsem[2 * p])
            pltpu.async_copy(norm2x.at[pl.ds(row0 + t * CB, CB)], nb,
                             gsem[2 * p + 1])
            pltpu.make_async_copy(agg.at[pl.ds(row0, CB)], db,
                                  gsem[2 * p]).wait()
            pltpu.make_async_copy(norm2x.at[pl.ds(row0, CB)], nb,
                                  gsem[2 * p + 1]).wait()

            def _srow(r, _):
                for j in range(H // L):
                    db[r, pl.ds(j * L, L)] = (
                        db[r, pl.ds(j * L, L)] * nb[r, pl.ds(j * L, L)]
                    )
                return 0

            lax.fori_loop(0, CB, _srow, 0)
            pltpu.async_copy(db, sout.at[pl.ds(coff + row0 + t * CB, CB)],
                             ssem[2 * p])
        for p in range(2):
            pltpu.make_async_copy(
                rows[2 * p], sout.at[pl.ds(coff, CB)], ssem[2 * p]).wait()

    souts = [o1, o2, o3, o4]
    sins = [s0] + souts[:3]
    for k in range(4):
        one_hop(sins[k], souts[k], garr, sarr)


_hops_call = functools.partial(
    pl.kernel,
    out_type=[jax.ShapeDtypeStruct((NC * NP, H), _f32) for _ in range(4)],
    mesh=plsc.VectorSubcoreMesh(core_axis_name="c", subcore_axis_name="s"),
    scratch_types=(
        [pltpu.VMEM_SHARED((NP, H), _f32)]
        + [pltpu.VMEM((CB,), _i32) for _ in range(8)]
        + [pltpu.VMEM((CB,), _i32) for _ in range(8)]
        + [pltpu.VMEM((CB, H), _f32) for _ in range(4)]
        + [pltpu.SemaphoreType.DMA for _ in range(16)]
    ),
)(_hops_body)


def _prep_body(feat_ref, deg2_ref, s0_ref, n2x_ref, rn_ref):
    d = deg2_ref[0:NP, :] + deg2_ref[NP : 2 * NP, :]
    cl = jnp.maximum(d, 1.0)
    norm = lax.rsqrt(cl)
    n2x_ref[...] = jnp.broadcast_to(1.0 / cl, (NP, H))
    rn_ref[...] = jnp.sqrt(cl)
    s0_ref[0:NP, :] = feat_ref[:, 0:H] * norm
    s0_ref[NP : 2 * NP, :] = feat_ref[:, H : 2 * H] * norm


_prep_call = pl.pallas_call(
    _prep_body,
    out_shape=[
        jax.ShapeDtypeStruct((NC * NP, H), _f32),
        jax.ShapeDtypeStruct((NP, H), _f32),
        jax.ShapeDtypeStruct((NP, 1), _f32),
    ],
)


def _mm_body(feat_ref, w0_ref, wh_ref, rn_ref, b_ref, *rest):
    s_refs = rest[:8]
    out_ref = rest[8]
    acc = jnp.zeros((RS, D), _f32)
    for k in range(8):
        for c in range(NC):
            acc = acc + jnp.dot(s_refs[k][c], wh_ref[k, c],
                                preferred_element_type=_f32)
    base = jnp.dot(feat_ref[...], w0_ref[...], preferred_element_type=_f32)
    out_ref[...] = base + rn_ref[...] * acc + b_ref[...]


_mm_call = pl.pallas_call(
    _mm_body,
    grid=(NS,),
    in_specs=[
        pl.BlockSpec((RS, D), lambda i: (i, 0)),
        pl.BlockSpec((D, D), lambda i: (0, 0)),
        pl.BlockSpec((8, NC, H, D), lambda i: (0, 0, 0, 0)),
        pl.BlockSpec((RS, 1), lambda i: (i, 0)),
        pl.BlockSpec((1, D), lambda i: (0, 0)),
    ] + [pl.BlockSpec((NC, RS, H), lambda i: (0, i, 0)) for _ in range(8)],
    out_specs=pl.BlockSpec((RS, D), lambda i: (i, 0)),
    out_shape=jax.ShapeDtypeStruct((NP, D), _f32),
)


def kernel(feat, edge_index, W, b):
    src = edge_index[0].astype(_i32)
    dst = edge_index[1].astype(_i32)

    pad = ET - E
    zpad = jnp.zeros((pad,), _i32)
    npad = jnp.full((pad,), N, _i32)
    dst_g = jnp.concatenate([dst, zpad]).reshape(NS * NCH, CB)
    dst_s = jnp.concatenate([dst, npad]).reshape(NS * NCH, CB)
    src_g = jnp.concatenate([src, zpad]).reshape(NS * NCH, CB)
    src_s = jnp.concatenate([src, npad]).reshape(NS * NCH, CB)
    dst_g2 = jnp.concatenate([dst_g, dst_g + NP], axis=0)
    src_g2 = jnp.concatenate([src_g, src_g + NP], axis=0)
    darr = dst_s.reshape(NC * NS, DNCH, DCB)

    featp = jnp.pad(feat, ((0, NP - N), (0, 0)))

    deg2 = _deg_call(darr)
    s0, n2x, rn = _prep_call(featp, deg2.reshape(NC * NP, 1))

    s_a = _hops_call(s0, dst_g2, src_s, n2x)
    s_b = _hops_call(s_a[3], src_g2, dst_s, n2x)
    s_list = list(s_a) + list(s_b)

    w0 = W[0:D]
    wh = W[D:].reshape(8, NC, H, D)
    b2 = b.reshape(1, D)
    s3d = [sk.reshape(NC, NP, H) for sk in s_list]
    outp = _mm_call(featp, w0, wh, rn, b2, *s3d)
    return outp[:N]
